# Initial kernel scaffold; baseline (speedup 1.0000x reference)
import jax
import jax.numpy as jnp
from jax.experimental import pallas as pl


def kernel(pred, target):
    raise NotImplementedError("write your pallas kernel here")


# speedup vs baseline: 38.9612x; 38.9612x over previous
import functools

import jax
import jax.numpy as jnp
from jax import lax
from jax.experimental import pallas as pl
from jax.experimental.pallas import tpu as pltpu
from jax.experimental.pallas import tpu_sc as plsc

NUM_BINS = 256
N_PIX = 512 * 512
N_CH = 16 * 3
LANES = 16
N_TILES = 32
CH_PER_TILE = 3
CHUNK = 16384
N_CHUNKS = N_PIX // CHUNK
UNROLL = 8
VREG_ITERS = CHUNK // (LANES * UNROLL)


def _hist_body(pred_hbm, target_hbm, out_hbm, buf, hist, outbuf):
    cid = lax.axis_index("c")
    sid = lax.axis_index("s")
    wid = sid * 2 + cid
    is_pred = wid < 16
    local = lax.rem(wid, 16)
    lane_base = lax.iota(jnp.int32, LANES) * NUM_BINS
    ones = jnp.ones((LANES,), jnp.float32)
    zeros = jnp.zeros((LANES,), jnp.float32)

    for j in range(CH_PER_TILE):
        ch = local * CH_PER_TILE + j
        base = ch * N_PIX

        def zero_body(k, carry):
            hist[pl.ds(k * LANES, LANES)] = zeros
            return carry

        lax.fori_loop(0, NUM_BINS, zero_body, 0)

        def chunk_body(cidx, carry):
            off = base + cidx * CHUNK

            @pl.when(is_pred)
            def _():
                pltpu.sync_copy(pred_hbm.at[pl.ds(off, CHUNK)], buf)

            @pl.when(jnp.logical_not(is_pred))
            def _():
                pltpu.sync_copy(target_hbm.at[pl.ds(off, CHUNK)], buf)

            def vec_body(i, c2):
                for u in range(UNROLL):
                    x = buf[pl.ds((i * UNROLL + u) * LANES, LANES)]
                    idx = (x * float(NUM_BINS)).astype(jnp.int32)
                    idx = jnp.clip(idx, 0, NUM_BINS - 1)
                    plsc.addupdate_scatter(hist, [idx + lane_base], ones)
                return c2

            lax.fori_loop(0, VREG_ITERS, vec_body, 0)
            return carry

        lax.fori_loop(0, N_CHUNKS, chunk_body, 0)

        def red_body(g, carry):
            acc = hist[pl.ds(g * LANES, LANES)]
            for l in range(1, LANES):
                acc = acc + hist[pl.ds(l * NUM_BINS + g * LANES, LANES)]
            outbuf[pl.ds(g * LANES, LANES)] = acc
            return carry

        lax.fori_loop(0, NUM_BINS // LANES, red_body, 0)

        u_row = jnp.where(is_pred, ch, N_CH + ch)
        pltpu.sync_copy(outbuf, out_hbm.at[pl.ds(u_row * NUM_BINS, NUM_BINS)])


@functools.partial(
    pl.kernel,
    mesh=plsc.VectorSubcoreMesh(core_axis_name="c", subcore_axis_name="s"),
    out_type=jax.ShapeDtypeStruct((2 * N_CH * NUM_BINS,), jnp.float32),
    scratch_types=[
        pltpu.VMEM((CHUNK,), jnp.float32),
        pltpu.VMEM((LANES * NUM_BINS,), jnp.float32),
        pltpu.VMEM((NUM_BINS,), jnp.float32),
    ],
    compiler_params=pltpu.CompilerParams(needs_layout_passes=False),
)
def _hist_kernel(pred_hbm, target_hbm, out_hbm, buf, hist, outbuf):
    _hist_body(pred_hbm, target_hbm, out_hbm, buf, hist, outbuf)


def kernel(pred, target):
    hist = _hist_kernel(pred.reshape(-1), target.reshape(-1))
    hist = hist.reshape(2, N_CH, NUM_BINS)
    p = hist[0] / (hist[0].sum(axis=1, keepdims=True) + 1e-8)
    t = hist[1] / (hist[1].sum(axis=1, keepdims=True) + 1e-8)
    return jnp.mean((p - t) ** 2)


# speedup vs baseline: 46.5364x; 1.1944x over previous
import functools

import jax
import jax.numpy as jnp
from jax import lax
from jax.experimental import pallas as pl
from jax.experimental.pallas import tpu as pltpu
from jax.experimental.pallas import tpu_sc as plsc

NUM_BINS = 256
N_PIX = 512 * 512
N_CH = 16 * 3
LANES = 16
CH_PER_TILE = 3
CHUNK = 16384
N_CHUNKS = N_PIX // CHUNK
VREGS = CHUNK // LANES


def _hist_body(pred_hbm, target_hbm, out_hbm, buf0, buf1, hist, outbuf,
               sem0, sem1):
    cid = lax.axis_index("c")
    sid = lax.axis_index("s")
    wid = sid * 2 + cid
    is_pred = wid < 16
    local = lax.rem(wid, 16)
    lane = lax.iota(jnp.int32, LANES)
    ones = jnp.ones((LANES,), jnp.float32)
    zeros = jnp.zeros((LANES,), jnp.float32)
    bufs = (buf0, buf1)
    sems = (sem0, sem1)

    def start_copy(off, b):
        @pl.when(is_pred)
        def _():
            pltpu.async_copy(pred_hbm.at[pl.ds(off, CHUNK)], bufs[b], sems[b])

        @pl.when(jnp.logical_not(is_pred))
        def _():
            pltpu.async_copy(target_hbm.at[pl.ds(off, CHUNK)], bufs[b],
                             sems[b])

    def wait_copy(b):
        pltpu.make_async_copy(pred_hbm.at[pl.ds(0, CHUNK)], bufs[b],
                              sems[b]).wait()

    for j in range(CH_PER_TILE):
        ch = local * CH_PER_TILE + j
        base = ch * N_PIX

        def zero_body(k, carry):
            hist[pl.ds(k * LANES, LANES)] = zeros
            return carry

        lax.fori_loop(0, NUM_BINS, zero_body, 0, unroll=8)

        start_copy(base, 0)

        def pair_body(i2, carry):
            for b in range(2):
                cidx = i2 * 2 + b

                @pl.when(cidx + 1 < N_CHUNKS)
                def _():
                    start_copy(base + (cidx + 1) * CHUNK, 1 - b)

                wait_copy(b)
                buf = bufs[b]

                def vec_body(i, c2):
                    x = buf[pl.ds(i * LANES, LANES)]
                    off = (x * 4096.0).astype(jnp.int32)
                    off = (off & 0xFF0) | lane
                    plsc.addupdate_scatter(hist, [off], ones)
                    return c2

                lax.fori_loop(0, VREGS, vec_body, 0, unroll=8)
            return carry

        lax.fori_loop(0, N_CHUNKS // 2, pair_body, 0)

        def red_body(g, carry):
            row = hist[pl.ds(g * LANES, LANES)]
            s = jnp.sum(row)
            plsc.store_scatter(outbuf, [jnp.broadcast_to(g, (LANES,))],
                               jnp.broadcast_to(s, (LANES,)),
                               mask=lane == 0)
            return carry

        lax.fori_loop(0, NUM_BINS, red_body, 0, unroll=4)

        u_row = jnp.where(is_pred, ch, N_CH + ch)
        pltpu.sync_copy(outbuf, out_hbm.at[pl.ds(u_row * NUM_BINS, NUM_BINS)])


@functools.partial(
    pl.kernel,
    mesh=plsc.VectorSubcoreMesh(core_axis_name="c", subcore_axis_name="s"),
    out_type=jax.ShapeDtypeStruct((2 * N_CH * NUM_BINS,), jnp.float32),
    scratch_types=[
        pltpu.VMEM((CHUNK,), jnp.float32),
        pltpu.VMEM((CHUNK,), jnp.float32),
        pltpu.VMEM((NUM_BINS * LANES,), jnp.float32),
        pltpu.VMEM((NUM_BINS,), jnp.float32),
        pltpu.SemaphoreType.DMA,
        pltpu.SemaphoreType.DMA,
    ],
    compiler_params=pltpu.CompilerParams(needs_layout_passes=False),
)
def _hist_kernel(pred_hbm, target_hbm, out_hbm, buf0, buf1, hist, outbuf,
                 sem0, sem1):
    _hist_body(pred_hbm, target_hbm, out_hbm, buf0, buf1, hist, outbuf,
               sem0, sem1)


def kernel(pred, target):
    hist = _hist_kernel(pred.reshape(-1), target.reshape(-1))
    hist = hist.reshape(2, N_CH, NUM_BINS)
    p = hist[0] / (hist[0].sum(axis=1, keepdims=True) + 1e-8)
    t = hist[1] / (hist[1].sum(axis=1, keepdims=True) + 1e-8)
    return jnp.mean((p - t) ** 2)
